# Initial kernel scaffold; baseline (speedup 1.0000x reference)
import jax
import jax.numpy as jnp
from jax.experimental import pallas as pl


def kernel(x, weight, bias, type_vec):
    raise NotImplementedError("write your pallas kernel here")


# speedup vs baseline: 10.3665x; 10.3665x over previous
import functools

import jax
import jax.numpy as jnp
from jax import lax
from jax.experimental import pallas as pl
from jax.experimental.pallas import tpu as pltpu
from jax.experimental.pallas import tpu_sc as plsc

N = 320000
C = 128
T = 8
EPS = 1e-05

NC = 2
NS = 16
NW = NC * NS
R = N // NW
CH = 400
NCHUNK = R // CH
LANES = 16
G = C // LANES

_mesh = plsc.VectorSubcoreMesh(core_axis_name="c", subcore_axis_name="s")


@functools.partial(
    pl.kernel,
    out_type=jax.ShapeDtypeStruct((NW, 2, T, C), jnp.float32),
    mesh=_mesh,
    scratch_types=[
        pltpu.VMEM((CH, C), jnp.float32),
        pltpu.VMEM((16,), jnp.int32),
        pltpu.VMEM((2, T, C), jnp.float32),
    ],
)
def _pass1(x_hbm, bnd_hbm, out_hbm, xc, bnd, acc):
    wid = lax.axis_index("s") * NC + lax.axis_index("c")
    base = wid * R
    pltpu.sync_copy(bnd_hbm, bnd)
    bvec = bnd[...]
    zero = jnp.zeros((LANES,), jnp.float32)
    for t in range(T):
        for g in range(G):
            acc[0, t, pl.ds(g * LANES, LANES)] = zero
            acc[1, t, pl.ds(g * LANES, LANES)] = zero
    def _chunk(k, carry):
        c0 = pl.multiple_of(base + k * CH, 8)
        pltpu.sync_copy(x_hbm.at[pl.ds(c0, CH)], xc)
        for t in range(T):
            lo = jnp.maximum(bvec[t], c0) - c0
            hi = jnp.minimum(bvec[t + 1], c0 + CH) - c0

            @pl.when(hi > lo)
            def _accum(t=t, lo=lo, hi=hi):
                def body(r, carry):
                    out = []
                    for g in range(G):
                        v = xc[r, pl.ds(g * LANES, LANES)]
                        out.append(carry[g] + v)
                        out.append(carry[G + g] + v * v)
                    return tuple(out[0::2]) + tuple(out[1::2])

                init = (zero,) * (2 * G)
                res = lax.fori_loop(lo, hi, body, init)
                for g in range(G):
                    plsc.addupdate(acc.at[0, t, pl.ds(g * LANES, LANES)],
                                   res[g])
                    plsc.addupdate(acc.at[1, t, pl.ds(g * LANES, LANES)],
                                   res[G + g])

        return carry

    lax.fori_loop(0, NCHUNK, _chunk, 0)
    pltpu.sync_copy(acc, out_hbm.at[wid])


def _finalize_body(part_ref, bnd_ref, w_ref, b_ref, ss_ref):
    part = part_ref[...]
    s = jnp.sum(part.reshape(NW, 2 * T, C), axis=0)
    sums, sqs = s[:T], s[T:]
    bnd = bnd_ref[...]
    counts = (bnd[1:T + 1] - bnd[:T]).astype(jnp.float32)
    safe = jnp.maximum(counts, 1.0)
    mean = sums / safe
    var = sqs / safe - mean * mean
    inv = lax.rsqrt(jnp.clip(var, EPS, None))
    scale = inv * w_ref[...]
    shift = b_ref[...] - mean * scale
    ss_ref[...] = jnp.concatenate([scale, shift], axis=0)


_finalize = pl.pallas_call(
    _finalize_body,
    out_shape=jax.ShapeDtypeStruct((2 * T, C), jnp.float32),
)


@functools.partial(
    pl.kernel,
    out_type=jax.ShapeDtypeStruct((N, C), jnp.float32),
    mesh=_mesh,
    scratch_types=[
        pltpu.VMEM((CH, C), jnp.float32),
        pltpu.VMEM((16,), jnp.int32),
        pltpu.VMEM((2, T, C), jnp.float32),
    ],
)
def _pass2(x_hbm, bnd_hbm, ss_hbm, out_hbm, xc, bnd, ss):
    wid = lax.axis_index("s") * NC + lax.axis_index("c")
    base = wid * R
    pltpu.sync_copy(bnd_hbm, bnd)
    pltpu.sync_copy(ss_hbm, ss)
    bvec = bnd[...]
    def _chunk(k, carry):
        c0 = pl.multiple_of(base + k * CH, 8)
        pltpu.sync_copy(x_hbm.at[pl.ds(c0, CH)], xc)
        for t in range(T):
            lo = jnp.maximum(bvec[t], c0) - c0
            hi = jnp.minimum(bvec[t + 1], c0 + CH) - c0

            @pl.when(hi > lo)
            def _norm(t=t, lo=lo, hi=hi):
                sc = [ss[0, t, pl.ds(g * LANES, LANES)] for g in range(G)]
                sh = [ss[1, t, pl.ds(g * LANES, LANES)] for g in range(G)]

                def body(r, carry):
                    for g in range(G):
                        xc[r, pl.ds(g * LANES, LANES)] = (
                            xc[r, pl.ds(g * LANES, LANES)] * sc[g] + sh[g])
                    return carry

                lax.fori_loop(lo, hi, body, 0)

        pltpu.sync_copy(xc, out_hbm.at[pl.ds(c0, CH)])
        return carry

    lax.fori_loop(0, NCHUNK, _chunk, 0)


def kernel(x, weight, bias, type_vec):
    bnd = jnp.searchsorted(
        type_vec, jnp.arange(T + 1, dtype=jnp.int32), side="left"
    ).astype(jnp.int32)
    bnd16 = jnp.concatenate(
        [bnd, jnp.full((16 - (T + 1),), N, jnp.int32)])
    partials = _pass1(x, bnd16)
    ss = _finalize(partials.reshape(NW * 2 * T, C), bnd16.reshape(16, 1),
                   weight, bias)
    return _pass2(x, bnd16, ss.reshape(2, T, C))


# speedup vs baseline: 13.4570x; 1.2981x over previous
import functools

import jax
import jax.numpy as jnp
from jax import lax
from jax.experimental import pallas as pl
from jax.experimental.pallas import tpu as pltpu
from jax.experimental.pallas import tpu_sc as plsc

N = 320000
C = 128
T = 8
EPS = 1e-05

NC = 2
NS = 16
NW = NC * NS
R = N // NW
CH = 400
NCHUNK = R // CH
LANES = 16
G = C // LANES

_mesh = plsc.VectorSubcoreMesh(core_axis_name="c", subcore_axis_name="s")


@functools.partial(
    pl.kernel,
    out_type=jax.ShapeDtypeStruct((NW, 2, T, C), jnp.float32),
    mesh=_mesh,
    scratch_types=[
        pltpu.VMEM((CH, C), jnp.float32),
        pltpu.VMEM((CH, C), jnp.float32),
        pltpu.VMEM((16,), jnp.int32),
        pltpu.VMEM((2, T, C), jnp.float32),
        pltpu.SemaphoreType.DMA,
        pltpu.SemaphoreType.DMA,
    ],
)
def _pass1(x_hbm, bnd_hbm, out_hbm, xc0, xc1, bnd, acc, sem0, sem1):
    wid = lax.axis_index("s") * NC + lax.axis_index("c")
    base = wid * R
    pltpu.sync_copy(bnd_hbm, bnd)
    bvec = bnd[...]
    zero = jnp.zeros((LANES,), jnp.float32)
    for t in range(T):
        for g in range(G):
            acc[0, t, pl.ds(g * LANES, LANES)] = zero
            acc[1, t, pl.ds(g * LANES, LANES)] = zero

    bufs = (xc0, xc1)
    sems = (sem0, sem1)

    def _src(k):
        c0 = pl.multiple_of(base + k * CH, 8)
        return x_hbm.at[pl.ds(c0, CH)]

    def _start_in(k, b):
        pltpu.async_copy(_src(k), bufs[b], sems[b])

    def _wait_in(k, b):
        pltpu.make_async_copy(_src(k), bufs[b], sems[b]).wait()

    def _accum_chunk(k, b):
        xc = bufs[b]
        c0 = base + k * CH
        for t in range(T):
            lo = jnp.maximum(bvec[t], c0) - c0
            hi = jnp.minimum(bvec[t + 1], c0 + CH) - c0

            @pl.when(hi > lo)
            def _accum(t=t, lo=lo, hi=hi, xc=xc):
                def body(r, carry):
                    out = []
                    for g in range(G):
                        v = xc[r, pl.ds(g * LANES, LANES)]
                        out.append(carry[g] + v)
                        out.append(carry[G + g] + v * v)
                    return tuple(out[0::2]) + tuple(out[1::2])

                init = (zero,) * (2 * G)
                res = lax.fori_loop(lo, hi, body, init)
                for g in range(G):
                    plsc.addupdate(acc.at[0, t, pl.ds(g * LANES, LANES)],
                                   res[g])
                    plsc.addupdate(acc.at[1, t, pl.ds(g * LANES, LANES)],
                                   res[G + g])

    _start_in(0, 0)

    def _pair(j, carry):
        k0 = 2 * j
        _wait_in(k0, 0)
        _start_in(k0 + 1, 1)
        _accum_chunk(k0, 0)
        _wait_in(k0 + 1, 1)
        _start_in(k0 + 2, 0)
        _accum_chunk(k0 + 1, 1)
        return carry

    lax.fori_loop(0, NCHUNK // 2, _pair, 0)
    _wait_in(NCHUNK - 1, 0)
    _accum_chunk(NCHUNK - 1, 0)
    pltpu.sync_copy(acc, out_hbm.at[wid])


def _finalize_body(part_ref, bnd_ref, w_ref, b_ref, ss_ref):
    part = part_ref[...]
    s = jnp.sum(part.reshape(NW, 2 * T, C), axis=0)
    sums, sqs = s[:T], s[T:]
    bnd = bnd_ref[...]
    counts = (bnd[1:T + 1] - bnd[:T]).astype(jnp.float32)
    safe = jnp.maximum(counts, 1.0)
    mean = sums / safe
    var = sqs / safe - mean * mean
    inv = lax.rsqrt(jnp.clip(var, EPS, None))
    scale = inv * w_ref[...]
    shift = b_ref[...] - mean * scale
    ss_ref[...] = jnp.concatenate([scale, shift], axis=0)


_finalize = pl.pallas_call(
    _finalize_body,
    out_shape=jax.ShapeDtypeStruct((2 * T, C), jnp.float32),
)


@functools.partial(
    pl.kernel,
    out_type=jax.ShapeDtypeStruct((N, C), jnp.float32),
    mesh=_mesh,
    scratch_types=[
        pltpu.VMEM((CH, C), jnp.float32),
        pltpu.VMEM((CH, C), jnp.float32),
        pltpu.VMEM((16,), jnp.int32),
        pltpu.VMEM((2, T, C), jnp.float32),
        pltpu.SemaphoreType.DMA,
        pltpu.SemaphoreType.DMA,
        pltpu.SemaphoreType.DMA,
        pltpu.SemaphoreType.DMA,
    ],
)
def _pass2(x_hbm, bnd_hbm, ss_hbm, out_hbm, xc0, xc1, bnd, ss,
           si0, si1, so0, so1):
    wid = lax.axis_index("s") * NC + lax.axis_index("c")
    base = wid * R
    pltpu.sync_copy(bnd_hbm, bnd)
    pltpu.sync_copy(ss_hbm, ss)
    bvec = bnd[...]

    bufs = (xc0, xc1)
    isems = (si0, si1)
    osems = (so0, so1)

    def _hslice(ref, k):
        c0 = pl.multiple_of(base + k * CH, 8)
        return ref.at[pl.ds(c0, CH)]

    def _start_in(k, b):
        pltpu.async_copy(_hslice(x_hbm, k), bufs[b], isems[b])

    def _wait_in(k, b):
        pltpu.make_async_copy(_hslice(x_hbm, k), bufs[b], isems[b]).wait()

    def _start_out(k, b):
        pltpu.async_copy(bufs[b], _hslice(out_hbm, k), osems[b])

    def _wait_out(k, b):
        pltpu.make_async_copy(bufs[b], _hslice(out_hbm, k), osems[b]).wait()

    def _norm_chunk(k, b):
        xc = bufs[b]
        c0 = base + k * CH
        for t in range(T):
            lo = jnp.maximum(bvec[t], c0) - c0
            hi = jnp.minimum(bvec[t + 1], c0 + CH) - c0

            @pl.when(hi > lo)
            def _norm(t=t, lo=lo, hi=hi, xc=xc):
                sc = [ss[0, t, pl.ds(g * LANES, LANES)] for g in range(G)]
                sh = [ss[1, t, pl.ds(g * LANES, LANES)] for g in range(G)]

                def body(r, carry):
                    for g in range(G):
                        xc[r, pl.ds(g * LANES, LANES)] = (
                            xc[r, pl.ds(g * LANES, LANES)] * sc[g] + sh[g])
                    return carry

                lax.fori_loop(lo, hi, body, 0)

    _start_in(0, 0)

    def _pair(j, carry):
        k0 = 2 * j
        _wait_in(k0, 0)
        _norm_chunk(k0, 0)
        _start_out(k0, 0)

        @pl.when(j > 0)
        def _():
            _wait_out(k0 - 1, 1)

        _start_in(k0 + 1, 1)
        _wait_in(k0 + 1, 1)
        _norm_chunk(k0 + 1, 1)
        _start_out(k0 + 1, 1)
        _wait_out(k0, 0)
        _start_in(k0 + 2, 0)
        return carry

    lax.fori_loop(0, NCHUNK // 2, _pair, 0)
    _wait_in(NCHUNK - 1, 0)
    _norm_chunk(NCHUNK - 1, 0)
    _start_out(NCHUNK - 1, 0)
    _wait_out(NCHUNK - 2, 1)
    _wait_out(NCHUNK - 1, 0)


def kernel(x, weight, bias, type_vec):
    bnd = jnp.searchsorted(
        type_vec, jnp.arange(T + 1, dtype=jnp.int32), side="left"
    ).astype(jnp.int32)
    bnd16 = jnp.concatenate(
        [bnd, jnp.full((16 - (T + 1),), N, jnp.int32)])
    partials = _pass1(x, bnd16)
    ss = _finalize(partials.reshape(NW * 2 * T, C), bnd16.reshape(16, 1),
                   weight, bias)
    return _pass2(x, bnd16, ss.reshape(2, T, C))
